# baseline (speedup 1.0000x reference)
import jax
import jax.numpy as jnp
from jax.experimental import pallas as pl
from jax.experimental.pallas import tpu as pltpu

TI = 512
N = 4096
NI = N // TI


def _update(self_rows, agg, deg, w_ref, b_ref):
    agg = jnp.where(deg > 0, agg / jnp.maximum(deg, 1.0), 0.0)
    inp = jnp.concatenate([self_rows, agg], axis=1)
    h = jax.nn.sigmoid(
        jax.lax.dot_general(inp, w_ref[...],
                            (((1,), (1,)), ((), ())),
                            preferred_element_type=jnp.float32)
        + b_ref[...]
    )
    norm = jnp.sqrt(jnp.sum(h * h, axis=1, keepdims=True))
    return h / (norm + 1e-6)


def _body(adj_ref, feat_ref, featb_ref, w0_ref, b0_ref, w1_ref, b1_ref,
          wd_ref, bd_ref, lab_ref,
          a8_ref, out0_ref, out0b_ref, acc1_ref, deg_ref):
    k = pl.program_id(1)
    base = k * TI

    @pl.when(k < NI)
    def _main():
        a_i32 = adj_ref[0]
        abf = a_i32.astype(jnp.bfloat16)
        a8_ref[pl.ds(base, TI), :] = a_i32.astype(jnp.int8)
        deg = jnp.sum(a_i32, axis=1).astype(jnp.float32)[:, None]
        deg_ref[pl.ds(base, TI), :] = deg
        agg = jax.lax.dot_general(
            abf, featb_ref[0],
            (((1,), (0,)), ((), ())),
            preferred_element_type=jnp.float32,
        )
        out0 = _update(feat_ref[0, pl.ds(base, TI), :], agg, deg,
                       w0_ref, b0_ref)
        out0_ref[pl.ds(base, TI), :] = out0
        out0b = out0.astype(jnp.bfloat16)
        out0b_ref[pl.ds(base, TI), :] = out0b

        iota = jax.lax.broadcasted_iota(jnp.int32, (N, 1), 0)
        out0b_m = jnp.where(iota < base + TI, out0b_ref[...],
                            jnp.bfloat16(0.0))
        acc1_ref[pl.ds(base, TI), :] = jax.lax.dot_general(
            abf, out0b_m,
            (((1,), (0,)), ((), ())),
            preferred_element_type=jnp.float32,
        )

        for j in range(NI - 1):
            @pl.when(j < k)
            def _col_strip(j=j):
                jb = j * TI
                blk = a8_ref[pl.ds(jb, TI),
                             pl.ds(base, TI)].astype(jnp.bfloat16)
                contrib = jax.lax.dot_general(
                    blk, out0b,
                    (((1,), (0,)), ((), ())),
                    preferred_element_type=jnp.float32,
                )
                acc1_ref[pl.ds(jb, TI), :] += contrib

    @pl.when(k == NI)
    def _epilogue():
        def upd(j, carry):
            jb = j * TI
            out1 = _update(out0_ref[pl.ds(jb, TI), :],
                           acc1_ref[pl.ds(jb, TI), :],
                           deg_ref[pl.ds(jb, TI), :],
                           w1_ref, b1_ref)
            lab_ref[0, pl.ds(jb, TI), :] = jax.nn.sigmoid(
                jax.lax.dot_general(out1, wd_ref[...],
                                    (((1,), (0,)), ((), ())),
                                    preferred_element_type=jnp.float32)
                + bd_ref[...]
            )
            return carry

        jax.lax.fori_loop(0, NI, upd, 0)


@jax.jit
def kernel(features, adj_matrix, W0, b0, W1, b1, Wd, bd):
    B, n, d = features.shape
    b0r = b0.reshape(1, -1)
    b1r = b1.reshape(1, -1)
    wdt = Wd.reshape(-1, 1)
    bdr = bd.reshape(1, 1)
    featb = features.astype(jnp.bfloat16)

    labels = pl.pallas_call(
        _body,
        grid=(B, NI + 1),
        in_specs=[
            pl.BlockSpec((1, TI, n),
                         lambda b, k: (b, jnp.minimum(k, NI - 1), 0)),
            pl.BlockSpec((1, n, d), lambda b, k: (b, 0, 0)),
            pl.BlockSpec((1, n, d), lambda b, k: (b, 0, 0)),
            pl.BlockSpec((d, 2 * d), lambda b, k: (0, 0)),
            pl.BlockSpec((1, d), lambda b, k: (0, 0)),
            pl.BlockSpec((d, 2 * d), lambda b, k: (0, 0)),
            pl.BlockSpec((1, d), lambda b, k: (0, 0)),
            pl.BlockSpec((d, 1), lambda b, k: (0, 0)),
            pl.BlockSpec((1, 1), lambda b, k: (0, 0)),
        ],
        out_specs=pl.BlockSpec((1, n, 1), lambda b, k: (b, 0, 0)),
        out_shape=jax.ShapeDtypeStruct((B, n, 1), jnp.float32),
        scratch_shapes=[
            pltpu.VMEM((n, n), jnp.int8),
            pltpu.VMEM((n, d), jnp.float32),
            pltpu.VMEM((n, d), jnp.bfloat16),
            pltpu.VMEM((n, d), jnp.float32),
            pltpu.VMEM((n, 1), jnp.float32),
        ],
        compiler_params=pltpu.CompilerParams(
            dimension_semantics=("arbitrary", "arbitrary"),
        ),
    )(adj_matrix, features, featb, W0, b0r, W1, b1r, wdt, bdr)

    return labels


# speedup vs baseline: 1.3319x; 1.3319x over previous
import jax
import jax.numpy as jnp
from jax.experimental import pallas as pl
from jax.experimental.pallas import tpu as pltpu

TI = 512
N = 4096
NI = N // TI


def _update(self_rows, agg, deg, w_ref, b_ref):
    agg = jnp.where(deg > 0, agg / jnp.maximum(deg, 1.0), 0.0)
    inp = jnp.concatenate([self_rows, agg], axis=1)
    h = jax.nn.sigmoid(
        jax.lax.dot_general(inp, w_ref[...],
                            (((1,), (1,)), ((), ())),
                            preferred_element_type=jnp.float32)
        + b_ref[...]
    )
    norm = jnp.sqrt(jnp.sum(h * h, axis=1, keepdims=True))
    return h / (norm + 1e-6)


def _body(adj_ref, featb_ref, w0_ref, b0_ref, w1_ref, b1_ref,
          wd_ref, bd_ref, lab_ref,
          a8_0, out0b_0, deg_0,
          a8_1, out0b_1, deg_1):
    p = pl.program_id(0)
    i = pl.program_id(1)
    base = i * TI

    def layer0(a8_ref, out0b_ref, deg_ref):
        a_i32 = adj_ref[0]
        a8_ref[pl.ds(base, TI), :] = a_i32.astype(jnp.int8)
        deg = jnp.sum(a_i32, axis=1).astype(jnp.float32)[:, None]
        deg_ref[pl.ds(base, TI), :] = deg
        agg = jax.lax.dot_general(
            a_i32.astype(jnp.bfloat16), featb_ref[0],
            (((1,), (0,)), ((), ())),
            preferred_element_type=jnp.float32,
        )
        self_rows = featb_ref[0, pl.ds(base, TI), :].astype(jnp.float32)
        out0 = _update(self_rows, agg, deg, w0_ref, b0_ref)
        out0b_ref[pl.ds(base, TI), :] = out0.astype(jnp.bfloat16)

    def layer1(a8_ref, out0b_ref, deg_ref):
        a = a8_ref[pl.ds(base, TI), :].astype(jnp.bfloat16)
        deg = deg_ref[pl.ds(base, TI), :]
        agg = jax.lax.dot_general(
            a, out0b_ref[...],
            (((1,), (0,)), ((), ())),
            preferred_element_type=jnp.float32,
        )
        self_rows = out0b_ref[pl.ds(base, TI), :].astype(jnp.float32)
        out1 = _update(self_rows, agg, deg, w1_ref, b1_ref)
        lab_ref[0] = jax.nn.sigmoid(
            jax.lax.dot_general(out1, wd_ref[...],
                                (((1,), (0,)), ((), ())),
                                preferred_element_type=jnp.float32)
            + bd_ref[...]
        )

    @pl.when(p == 0)
    def _p0():
        layer0(a8_0, out0b_0, deg_0)

    @pl.when(p == 1)
    def _p1():
        layer0(a8_1, out0b_1, deg_1)
        layer1(a8_0, out0b_0, deg_0)

    @pl.when(p == 2)
    def _p2():
        layer1(a8_1, out0b_1, deg_1)


@jax.jit
def kernel(features, adj_matrix, W0, b0, W1, b1, Wd, bd):
    B, n, d = features.shape
    b0r = b0.reshape(1, -1)
    b1r = b1.reshape(1, -1)
    wdt = Wd.reshape(-1, 1)
    bdr = bd.reshape(1, 1)
    featb = features.astype(jnp.bfloat16)

    labels = pl.pallas_call(
        _body,
        grid=(3, NI),
        in_specs=[
            pl.BlockSpec(
                (1, TI, n),
                lambda p, i: (jnp.minimum(p, 1),
                              jnp.where(p == 2, NI - 1, i), 0)),
            pl.BlockSpec((1, n, d), lambda p, i: (jnp.minimum(p, 1), 0, 0)),
            pl.BlockSpec((d, 2 * d), lambda p, i: (0, 0)),
            pl.BlockSpec((1, d), lambda p, i: (0, 0)),
            pl.BlockSpec((d, 2 * d), lambda p, i: (0, 0)),
            pl.BlockSpec((1, d), lambda p, i: (0, 0)),
            pl.BlockSpec((d, 1), lambda p, i: (0, 0)),
            pl.BlockSpec((1, 1), lambda p, i: (0, 0)),
        ],
        out_specs=pl.BlockSpec(
            (1, TI, 1), lambda p, i: (jnp.maximum(p - 1, 0), i, 0)),
        out_shape=jax.ShapeDtypeStruct((B, n, 1), jnp.float32),
        scratch_shapes=[
            pltpu.VMEM((n, n), jnp.int8),
            pltpu.VMEM((n, d), jnp.bfloat16),
            pltpu.VMEM((n, 1), jnp.float32),
            pltpu.VMEM((n, n), jnp.int8),
            pltpu.VMEM((n, d), jnp.bfloat16),
            pltpu.VMEM((n, 1), jnp.float32),
        ],
        compiler_params=pltpu.CompilerParams(
            dimension_semantics=("arbitrary", "arbitrary"),
        ),
    )(adj_matrix, featb, W0, b0r, W1, b1r, wdt, bdr)

    return labels


# speedup vs baseline: 1.5176x; 1.1395x over previous
import jax
import jax.numpy as jnp
from jax.experimental import pallas as pl
from jax.experimental.pallas import tpu as pltpu

TI = 512


def _update(self_rows, agg, deg, w_ref, b_ref):
    agg = jnp.where(deg > 0, agg / jnp.maximum(deg, 1.0), 0.0)
    inp = jnp.concatenate([self_rows, agg], axis=1)
    h = jax.nn.sigmoid(
        jax.lax.dot_general(inp, w_ref[...],
                            (((1,), (1,)), ((), ())),
                            preferred_element_type=jnp.float32)
        + b_ref[...]
    )
    norm = jnp.sqrt(jnp.sum(h * h, axis=1, keepdims=True))
    return h / (norm + 1e-6)


def _body(adj_ref, feat_ref, w0_ref, b0_ref, w1_ref, b1_ref,
          wd_ref, bd_ref, lab_ref,
          a8_ref, out0_ref, out0b_ref, deg_ref):
    p = pl.program_id(1)
    i = pl.program_id(2)
    base = i * TI

    @pl.when(p == 0)
    def _layer0():
        a_i32 = adj_ref[0]
        a8_ref[pl.ds(base, TI), :] = a_i32.astype(jnp.int8)
        deg = jnp.sum(a_i32, axis=1).astype(jnp.float32)[:, None]
        deg_ref[pl.ds(base, TI), :] = deg
        agg = jax.lax.dot_general(
            a_i32.astype(jnp.bfloat16), feat_ref[0].astype(jnp.bfloat16),
            (((1,), (0,)), ((), ())),
            preferred_element_type=jnp.float32,
        )
        out0 = _update(feat_ref[0, pl.ds(base, TI), :], agg, deg,
                       w0_ref, b0_ref)
        out0_ref[pl.ds(base, TI), :] = out0
        out0b_ref[pl.ds(base, TI), :] = out0.astype(jnp.bfloat16)

    @pl.when(p == 1)
    def _layer1():
        a = a8_ref[pl.ds(base, TI), :].astype(jnp.bfloat16)
        deg = deg_ref[pl.ds(base, TI), :]
        agg = jax.lax.dot_general(
            a, out0b_ref[...],
            (((1,), (0,)), ((), ())),
            preferred_element_type=jnp.float32,
        )
        out1 = _update(out0_ref[pl.ds(base, TI), :], agg, deg,
                       w1_ref, b1_ref)
        lab_ref[0] = jax.nn.sigmoid(
            jax.lax.dot_general(out1, wd_ref[...],
                                (((1,), (0,)), ((), ())),
                                preferred_element_type=jnp.float32)
            + bd_ref[...]
        )


@jax.jit
def kernel(features, adj_matrix, W0, b0, W1, b1, Wd, bd):
    B, n, d = features.shape
    ni = n // TI
    b0r = b0.reshape(1, -1)
    b1r = b1.reshape(1, -1)
    wdt = Wd.reshape(-1, 1)
    bdr = bd.reshape(1, 1)

    labels = pl.pallas_call(
        _body,
        grid=(B, 2, ni),
        in_specs=[
            pl.BlockSpec((1, TI, n),
                         lambda b, p, i: (b, jnp.where(p == 0, i, ni - 1), 0)),
            pl.BlockSpec((1, n, d), lambda b, p, i: (b, 0, 0)),
            pl.BlockSpec((d, 2 * d), lambda b, p, i: (0, 0)),
            pl.BlockSpec((1, d), lambda b, p, i: (0, 0)),
            pl.BlockSpec((d, 2 * d), lambda b, p, i: (0, 0)),
            pl.BlockSpec((1, d), lambda b, p, i: (0, 0)),
            pl.BlockSpec((d, 1), lambda b, p, i: (0, 0)),
            pl.BlockSpec((1, 1), lambda b, p, i: (0, 0)),
        ],
        out_specs=pl.BlockSpec((1, TI, 1), lambda b, p, i: (b, i, 0)),
        out_shape=jax.ShapeDtypeStruct((B, n, 1), jnp.float32),
        scratch_shapes=[
            pltpu.VMEM((n, n), jnp.int8),
            pltpu.VMEM((n, d), jnp.float32),
            pltpu.VMEM((n, d), jnp.bfloat16),
            pltpu.VMEM((n, 1), jnp.float32),
        ],
        compiler_params=pltpu.CompilerParams(
            dimension_semantics=("arbitrary", "arbitrary", "arbitrary"),
        ),
    )(adj_matrix, features, W0, b0r, W1, b1r, wdt, bdr)

    return labels


# speedup vs baseline: 1.5363x; 1.0123x over previous
import jax
import jax.numpy as jnp
from jax.experimental import pallas as pl
from jax.experimental.pallas import tpu as pltpu

TI = 1024


def _update(self_rows, agg, deg, w_ref, b_ref):
    agg = jnp.where(deg > 0, agg / jnp.maximum(deg, 1.0), 0.0)
    inp = jnp.concatenate([self_rows, agg], axis=1)
    h = jax.nn.sigmoid(
        jax.lax.dot_general(inp, w_ref[...],
                            (((1,), (1,)), ((), ())),
                            preferred_element_type=jnp.float32)
        + b_ref[...]
    )
    norm = jnp.sqrt(jnp.sum(h * h, axis=1, keepdims=True))
    return h / (norm + 1e-6)


def _body(adj_ref, feat_ref, w0_ref, b0_ref, w1_ref, b1_ref,
          wd_ref, bd_ref, lab_ref,
          a8_ref, out0_ref, out0b_ref, deg_ref):
    p = pl.program_id(1)
    i = pl.program_id(2)
    base = i * TI

    @pl.when(p == 0)
    def _layer0():
        a_i32 = adj_ref[0]
        a8_ref[pl.ds(base, TI), :] = a_i32.astype(jnp.int8)
        deg = jnp.sum(a_i32, axis=1).astype(jnp.float32)[:, None]
        deg_ref[pl.ds(base, TI), :] = deg
        agg = jax.lax.dot_general(
            a_i32.astype(jnp.bfloat16), feat_ref[0].astype(jnp.bfloat16),
            (((1,), (0,)), ((), ())),
            preferred_element_type=jnp.float32,
        )
        out0 = _update(feat_ref[0, pl.ds(base, TI), :], agg, deg,
                       w0_ref, b0_ref)
        out0_ref[pl.ds(base, TI), :] = out0
        out0b_ref[pl.ds(base, TI), :] = out0.astype(jnp.bfloat16)

    @pl.when(p == 1)
    def _layer1():
        a = a8_ref[pl.ds(base, TI), :].astype(jnp.bfloat16)
        deg = deg_ref[pl.ds(base, TI), :]
        agg = jax.lax.dot_general(
            a, out0b_ref[...],
            (((1,), (0,)), ((), ())),
            preferred_element_type=jnp.float32,
        )
        out1 = _update(out0_ref[pl.ds(base, TI), :], agg, deg,
                       w1_ref, b1_ref)
        lab_ref[0] = jax.nn.sigmoid(
            jax.lax.dot_general(out1, wd_ref[...],
                                (((1,), (0,)), ((), ())),
                                preferred_element_type=jnp.float32)
            + bd_ref[...]
        )


@jax.jit
def kernel(features, adj_matrix, W0, b0, W1, b1, Wd, bd):
    B, n, d = features.shape
    ni = n // TI
    b0r = b0.reshape(1, -1)
    b1r = b1.reshape(1, -1)
    wdt = Wd.reshape(-1, 1)
    bdr = bd.reshape(1, 1)

    labels = pl.pallas_call(
        _body,
        grid=(B, 2, ni),
        in_specs=[
            pl.BlockSpec((1, TI, n),
                         lambda b, p, i: (b, jnp.where(p == 0, i, ni - 1), 0)),
            pl.BlockSpec((1, n, d), lambda b, p, i: (b, 0, 0)),
            pl.BlockSpec((d, 2 * d), lambda b, p, i: (0, 0)),
            pl.BlockSpec((1, d), lambda b, p, i: (0, 0)),
            pl.BlockSpec((d, 2 * d), lambda b, p, i: (0, 0)),
            pl.BlockSpec((1, d), lambda b, p, i: (0, 0)),
            pl.BlockSpec((d, 1), lambda b, p, i: (0, 0)),
            pl.BlockSpec((1, 1), lambda b, p, i: (0, 0)),
        ],
        out_specs=pl.BlockSpec((1, TI, 1), lambda b, p, i: (b, i, 0)),
        out_shape=jax.ShapeDtypeStruct((B, n, 1), jnp.float32),
        scratch_shapes=[
            pltpu.VMEM((n, n), jnp.int8),
            pltpu.VMEM((n, d), jnp.float32),
            pltpu.VMEM((n, d), jnp.bfloat16),
            pltpu.VMEM((n, 1), jnp.float32),
        ],
        compiler_params=pltpu.CompilerParams(
            dimension_semantics=("arbitrary", "arbitrary", "arbitrary"),
            vmem_limit_bytes=110 * 1024 * 1024,
        ),
    )(adj_matrix, features, W0, b0r, W1, b1r, wdt, bdr)

    return labels
